# Initial kernel scaffold; baseline (speedup 1.0000x reference)
import jax
import jax.numpy as jnp
from jax.experimental import pallas as pl


def kernel(Z, z_to_index):
    raise NotImplementedError("write your pallas kernel here")


# speedup vs baseline: 5.2371x; 5.2371x over previous
import functools

import jax
import jax.numpy as jnp
from jax import lax
from jax.experimental import pallas as pl
from jax.experimental.pallas import tpu as pltpu
from jax.experimental.pallas import tpu_sc as plsc

_N = 100000
_C = 100
_NW = 32
_ROWS_W = 3136
_CHUNK = 448
_GROUPS = _CHUNK // 16
_TAIL = 96
_TAIL_GROUPS = _TAIL // 16

_mesh = plsc.VectorSubcoreMesh(core_axis_name="c", subcore_axis_name="s")


@functools.partial(
    pl.kernel,
    out_type=jax.ShapeDtypeStruct((_N, _C), jnp.float32),
    mesh=_mesh,
    scratch_types=[
        pltpu.VMEM((_ROWS_W,), jnp.int32),
        pltpu.VMEM((128,), jnp.int32),
        pltpu.VMEM((_CHUNK, _C), jnp.float32),
        pltpu.VMEM((_CHUNK, _C), jnp.float32),
        pltpu.VMEM((_CHUNK,), jnp.int32),
        pltpu.VMEM((_CHUNK,), jnp.int32),
        pltpu.SemaphoreType.DMA,
        pltpu.SemaphoreType.DMA,
    ],
    compiler_params=pltpu.CompilerParams(needs_layout_passes=False),
)
def _onehot_sc(zp_hbm, tab_hbm, zero_hbm, out_hbm,
               zbuf, tabv, buf0, buf1, sv0, sv1, sem0, sem1):
    wid = lax.axis_index("s") * 2 + lax.axis_index("c")
    base = wid * _ROWS_W

    pltpu.sync_copy(zp_hbm.at[pl.ds(base, _ROWS_W)], zbuf)
    pltpu.sync_copy(tab_hbm, tabv)
    cp0 = pltpu.async_copy(zero_hbm, buf0, sem0)
    cp1 = pltpu.async_copy(zero_hbm, buf1, sem1)
    cp0.wait()
    cp1.wait()

    lanes = lax.broadcasted_iota(jnp.int32, (16,), 0)
    ones = jnp.ones((16,), jnp.float32)
    zeros = jnp.zeros((16,), jnp.float32)

    bufs = (buf0, buf1)
    saves = (sv0, sv1)
    sems = (sem0, sem1)

    def out_dst(c, nrows=_CHUNK):
        return out_hbm.at[pl.ds(base + c * _CHUNK, nrows), :]

    def set_chunk(c, buf, sv, ngroups):
        for g in range(ngroups):
            z = zbuf[pl.ds(c * _CHUNK + 16 * g, 16)]
            idx = plsc.load_gather(tabv, [z])
            rows = lanes + (16 * g)
            plsc.store_scatter(buf, [rows, idx], ones)
            sv[pl.ds(16 * g, 16)] = idx

    def clear_chunk(buf, sv):
        for g in range(_GROUPS):
            idx = sv[pl.ds(16 * g, 16)]
            rows = lanes + (16 * g)
            plsc.store_scatter(buf, [rows, idx], zeros)

    for c in range(6):
        b = c % 2
        if c >= 2:
            pltpu.make_async_copy(bufs[b], out_dst(c - 2), sems[b]).wait()
            clear_chunk(bufs[b], saves[b])
        set_chunk(c, bufs[b], saves[b], _GROUPS)
        pltpu.async_copy(bufs[b], out_dst(c), sems[b])

    is_last = wid == (_NW - 1)

    @pl.when(jnp.logical_not(is_last))
    def _():
        pltpu.make_async_copy(bufs[0], out_dst(4), sems[0]).wait()
        clear_chunk(bufs[0], saves[0])
        set_chunk(6, bufs[0], saves[0], _GROUPS)
        pltpu.async_copy(bufs[0], out_dst(6), sems[0])
        pltpu.make_async_copy(bufs[1], out_dst(5), sems[1]).wait()
        pltpu.make_async_copy(bufs[0], out_dst(6), sems[0]).wait()

    @pl.when(is_last)
    def _():
        pltpu.make_async_copy(bufs[0], out_dst(4), sems[0]).wait()
        clear_chunk(bufs[0], saves[0])
        set_chunk(6, bufs[0], saves[0], _TAIL_GROUPS)
        pltpu.async_copy(bufs[0].at[pl.ds(0, _TAIL), :], out_dst(6, _TAIL), sems[0])
        pltpu.make_async_copy(bufs[1], out_dst(5), sems[1]).wait()
        pltpu.make_async_copy(bufs[0].at[pl.ds(0, _TAIL), :], out_dst(6, _TAIL), sems[0]).wait()


def kernel(Z, z_to_index):
    zp = jnp.pad(Z, (0, _NW * _ROWS_W - _N))
    tab = jnp.pad(z_to_index, (0, 128 - z_to_index.shape[0]))
    zero = jnp.zeros((_CHUNK, _C), jnp.float32)
    return _onehot_sc(zp, tab, zero)
